# baseline (speedup 1.0000x reference)
import functools

import jax
import jax.numpy as jnp
from jax import lax
from jax.experimental import pallas as pl
from jax.experimental.pallas import tpu as pltpu
from jax.experimental.pallas import tpu_sc as plsc

B = 16384
D = 64
NC = 2
NS = 16
NW = NC * NS
BPW = B // NW
L = 16
NG = BPW // L

_mesh = plsc.VectorSubcoreMesh(core_axis_name="c", subcore_axis_name="s")


@functools.partial(
    pl.kernel,
    mesh=_mesh,
    out_type=(
        jax.ShapeDtypeStruct((B * D,), jnp.float32),
        jax.ShapeDtypeStruct((B * D,), jnp.float32),
    ),
    scratch_types=[
        pltpu.VMEM((BPW,), jnp.int32),
        pltpu.VMEM((BPW,), jnp.int32),
        pltpu.VMEM((BPW * D,), jnp.float32),
        pltpu.VMEM((BPW * D,), jnp.float32),
        pltpu.SemaphoreType.DMA,
        pltpu.SemaphoreType.DMA,
    ],
)
def _dual_gather(w_idx_hbm, c_idx_hbm, wt_hbm, ct_hbm, w_out, c_out,
                 widx_v, cidx_v, wrows_v, crows_v, sem_w, sem_c):
    wid = lax.axis_index("s") * NC + lax.axis_index("c")
    base = wid * BPW
    ob0 = pl.multiple_of(base, 8)
    pltpu.sync_copy(w_idx_hbm.at[pl.ds(ob0, BPW)], widx_v)
    pltpu.sync_copy(c_idx_hbm.at[pl.ds(ob0, BPW)], cidx_v)

    def fire(g, _):
        vw = widx_v[pl.ds(g * L, L)] * D
        vc = cidx_v[pl.ds(g * L, L)] * D
        for l in range(L):
            r = g * L + l
            ow = pl.multiple_of(vw[l], 8)
            oc = pl.multiple_of(vc[l], 8)
            od = pl.multiple_of(r * D, 8)
            pltpu.async_copy(
                wt_hbm.at[pl.ds(ow, D)], wrows_v.at[pl.ds(od, D)], sem_w)
            pltpu.async_copy(
                ct_hbm.at[pl.ds(oc, D)], crows_v.at[pl.ds(od, D)], sem_c)
        return 0

    lax.fori_loop(0, NG, fire, 0)

    def drain(j, _):
        pltpu.make_async_copy(
            wt_hbm.at[pl.ds(0, D)], wrows_v.at[pl.ds(0, D)], sem_w).wait()
        pltpu.make_async_copy(
            ct_hbm.at[pl.ds(0, D)], crows_v.at[pl.ds(0, D)], sem_c).wait()
        return 0

    lax.fori_loop(0, BPW, drain, 0)

    ob = pl.multiple_of(base * D, 8)
    pltpu.sync_copy(wrows_v, w_out.at[pl.ds(ob, BPW * D)])
    pltpu.sync_copy(crows_v, c_out.at[pl.ds(ob, BPW * D)])


def kernel(X, word_table, context_table):
    w = X[:, 0]
    c = X[:, 1]
    wt_flat = word_table.reshape(-1)
    ct_flat = context_table.reshape(-1)
    w_rows, c_rows = _dual_gather(w, c, wt_flat, ct_flat)
    return (w_rows.reshape(B, 1, D), c_rows.reshape(B, 1, D))


# speedup vs baseline: 1.2639x; 1.2639x over previous
import functools

import jax
import jax.numpy as jnp
from jax import lax
from jax.experimental import pallas as pl
from jax.experimental.pallas import tpu as pltpu
from jax.experimental.pallas import tpu_sc as plsc

B = 16384
D = 64
NC = 2
NS = 16
NW = NC * NS
BPW = B // NW
L = 16
CH = 128
CHR = 256
NCHK = BPW // CHR
NG = CHR // L

_mesh = plsc.VectorSubcoreMesh(core_axis_name="c", subcore_axis_name="s")


@functools.partial(
    pl.kernel,
    mesh=_mesh,
    out_type=jax.ShapeDtypeStruct((B, D), jnp.float32),
    scratch_types=[
        pltpu.VMEM((BPW,), jnp.int32),
        pltpu.VMEM((BPW,), jnp.int32),
        pltpu.VMEM((CHR, 2 * D), jnp.float32),
        pltpu.VMEM((CHR, D), jnp.float32),
        pltpu.SemaphoreType.DMA,
    ],
)
def _pair_gather(idxp_hbm, half_hbm, t128_hbm, out_hbm,
                 idxp_v, half_v, pairs_v, rows_v, sem):
    wid = lax.axis_index("s") * NC + lax.axis_index("c")
    base = wid * BPW
    pltpu.sync_copy(idxp_hbm.at[pl.ds(base, BPW)], idxp_v)
    pltpu.sync_copy(half_hbm.at[pl.ds(base, BPW)], half_v)

    def chunk(k, _):
        copies = []
        for j in range(CHR // CH):
            copies.append(pltpu.async_copy(
                t128_hbm.at[idxp_v.at[pl.ds(k * CHR + j * CH, CH)]],
                pairs_v.at[pl.ds(j * CH, CH)], sem))
        for cp in copies:
            cp.wait()

        def select(g, _):
            hv = half_v[pl.ds(k * CHR + g * L, L)]
            for l in range(L):
                r = g * L + l
                h = hv[l]
                for q in range(D // L):
                    lo = pairs_v[r, pl.ds(q * L, L)]
                    hi = pairs_v[r, pl.ds(D + q * L, L)]
                    rows_v[r, pl.ds(q * L, L)] = jnp.where(h > 0, hi, lo)
            return 0

        lax.fori_loop(0, NG, select, 0)
        pltpu.sync_copy(rows_v, out_hbm.at[pl.ds(base + k * CHR, CHR)])
        return 0

    lax.fori_loop(0, NCHK, chunk, 0)


@functools.partial(
    pl.kernel,
    mesh=_mesh,
    out_type=jax.ShapeDtypeStruct((B, D), jnp.float32),
    scratch_types=[
        pltpu.VMEM((BPW,), jnp.int32),
        pltpu.VMEM((CHR, D), jnp.float32),
        pltpu.SemaphoreType.DMA,
    ],
)
def _rowstream_gather(idx_hbm, t_hbm, out_hbm, idx_v, rows_v, sem):
    wid = lax.axis_index("s") * NC + lax.axis_index("c")
    base = wid * BPW
    pltpu.sync_copy(idx_hbm.at[pl.ds(base, BPW)], idx_v)

    def chunk(k, _):
        def fire(g, _):
            vi = idx_v[pl.ds(k * CHR + g * L, L)]
            for l in range(L):
                pltpu.async_copy(
                    t_hbm.at[pl.ds(vi[l], 1)],
                    rows_v.at[pl.ds(g * L + l, 1)], sem)
            return 0

        lax.fori_loop(0, NG, fire, 0)

        def drain(j, _):
            pltpu.make_async_copy(
                t_hbm.at[pl.ds(0, 1)], rows_v.at[pl.ds(0, 1)], sem).wait()
            return 0

        lax.fori_loop(0, CHR, drain, 0)

        pltpu.sync_copy(rows_v, out_hbm.at[pl.ds(base + k * CHR, CHR)])
        return 0

    lax.fori_loop(0, NCHK, chunk, 0)


def kernel(X, word_table, context_table):
    w = X[:, 0]
    c = X[:, 1]
    cp = c // 2
    ch = c % 2
    ct128 = context_table.reshape(500001, 2 * D)
    w_rows = _rowstream_gather(w, word_table)
    c_rows = _pair_gather(cp, ch, ct128)
    return (w_rows[:, None, :], c_rows[:, None, :])


# speedup vs baseline: 1.5717x; 1.2436x over previous
import functools

import jax
import jax.numpy as jnp
from jax import lax
from jax.experimental import pallas as pl
from jax.experimental.pallas import tpu as pltpu
from jax.experimental.pallas import tpu_sc as plsc

B = 16384
D = 64
NC = 2
NS = 16
NW = NC * NS
BPW = B // NW
L = 16
CHR = 256
NCHK = BPW // CHR
NG = CHR // L

_mesh = plsc.VectorSubcoreMesh(core_axis_name="c", subcore_axis_name="s")


@functools.partial(
    pl.kernel,
    mesh=_mesh,
    out_type=(
        jax.ShapeDtypeStruct((B, D), jnp.float32),
        jax.ShapeDtypeStruct((B, D), jnp.float32),
    ),
    scratch_types=[
        pltpu.VMEM((BPW,), jnp.int32),
        pltpu.VMEM((BPW,), jnp.int32),
        pltpu.VMEM((CHR, D), jnp.float32),
        pltpu.VMEM((CHR, D), jnp.float32),
        pltpu.SemaphoreType.DMA,
        pltpu.SemaphoreType.DMA,
    ],
)
def _dual_gather(w_idx_hbm, c_idx_hbm, wt_hbm, ct_hbm, w_out, c_out,
                 widx_v, cidx_v, wrows_v, crows_v, sem_w, sem_c):
    wid = lax.axis_index("s") * NC + lax.axis_index("c")
    base = wid * BPW
    pltpu.sync_copy(w_idx_hbm.at[pl.ds(base, BPW)], widx_v)
    pltpu.sync_copy(c_idx_hbm.at[pl.ds(base, BPW)], cidx_v)

    def chunk(k, _):
        def fire(g, _):
            vw = widx_v[pl.ds(k * CHR + g * L, L)]
            vc = cidx_v[pl.ds(k * CHR + g * L, L)]
            for l in range(L):
                pltpu.async_copy(
                    wt_hbm.at[pl.ds(vw[l], 1)],
                    wrows_v.at[pl.ds(g * L + l, 1)], sem_w)
                pltpu.async_copy(
                    ct_hbm.at[pl.ds(vc[l], 1)],
                    crows_v.at[pl.ds(g * L + l, 1)], sem_c)
            return 0

        lax.fori_loop(0, NG, fire, 0)

        def drain(j, _):
            pltpu.make_async_copy(
                wt_hbm.at[pl.ds(0, 1)], wrows_v.at[pl.ds(0, 1)], sem_w).wait()
            pltpu.make_async_copy(
                ct_hbm.at[pl.ds(0, 1)], crows_v.at[pl.ds(0, 1)], sem_c).wait()
            return 0

        lax.fori_loop(0, CHR, drain, 0)

        pltpu.sync_copy(wrows_v, w_out.at[pl.ds(base + k * CHR, CHR)])
        pltpu.sync_copy(crows_v, c_out.at[pl.ds(base + k * CHR, CHR)])
        return 0

    lax.fori_loop(0, NCHK, chunk, 0)


def kernel(X, word_table, context_table):
    w = X[:, 0]
    c = X[:, 1]
    w_rows, c_rows = _dual_gather(w, c, word_table, context_table)
    return (w_rows[:, None, :], c_rows[:, None, :])
